# Initial kernel scaffold; baseline (speedup 1.0000x reference)
import jax
import jax.numpy as jnp
from jax.experimental import pallas as pl


def kernel(x, edge_index, edge_weight, batch, W1, b1, W2, b2, Wl, bl):
    raise NotImplementedError("write your pallas kernel here")


# speedup vs baseline: 8.0698x; 8.0698x over previous
import functools

import jax
import jax.numpy as jnp
from jax import lax
from jax.experimental import pallas as pl
from jax.experimental.pallas import tpu as pltpu
from jax.experimental.pallas import tpu_sc as plsc

N = 10000
E = 320000
D = 128
H = 256
C = 10
G = 64

NC = 2
NS = 16
HH = H // 2

NP = 10240
MB = 512
NMB = NP // MB

DEG_PAD = NP
DEG_PER_TILE = DEG_PAD // NS

SP_ROWS_T = 160
E_PAD = NS * SP_ROWS_T * 128
DEG_ROWS_W = E_PAD // (NC * NS * 128)
ROWS_PER_TILE = NP // NS


def _zeros16():
    return jnp.zeros((16,), jnp.float32)


def _deg_body(dst_hbm, ew_hbm, out_hbm, zb, dstb, ewb, acc, psem):
    c = lax.axis_index("c")
    s = lax.axis_index("s")
    w = c * NS + s
    row0 = w * DEG_ROWS_W
    p1 = pltpu.async_copy(dst_hbm.at[pl.ds(row0, DEG_ROWS_W)], dstb, psem)
    p2 = pltpu.async_copy(ew_hbm.at[pl.ds(row0, DEG_ROWS_W)], ewb, psem)
    for j in range(DEG_PER_TILE // 16):
        zb[pl.ds(j * 16, 16)] = _zeros16()
    pltpu.sync_copy(zb, acc.at[pl.ds(s * DEG_PER_TILE, DEG_PER_TILE)])
    p1.wait()
    p2.wait()
    plsc.subcore_barrier()

    def chunk(i, carry):
        pltpu.sync_copy(ewb.at[i], acc.at[dstb.at[i]], add=True)
        return carry

    lax.fori_loop(0, DEG_ROWS_W, chunk, 0)
    plsc.subcore_barrier()
    sl = pl.ds(s * DEG_PER_TILE, DEG_PER_TILE)
    pltpu.sync_copy(acc.at[sl], out_hbm.at[c, sl])


_deg_call = pl.kernel(
    _deg_body,
    out_type=jax.ShapeDtypeStruct((NC, DEG_PAD), jnp.float32),
    mesh=plsc.VectorSubcoreMesh(core_axis_name="c", subcore_axis_name="s",
                                num_cores=NC, num_subcores=NS),
    scratch_types=[
        pltpu.VMEM((DEG_PER_TILE,), jnp.float32),
        pltpu.VMEM((DEG_ROWS_W, 128), jnp.int32),
        pltpu.VMEM((DEG_ROWS_W, 128), jnp.float32),
        pltpu.VMEM_SHARED((DEG_PAD,), jnp.float32),
        pltpu.SemaphoreType.DMA,
    ],
    compiler_params=pltpu.CompilerParams(needs_layout_passes=False),
)


ACC_ROWS = 10000
TILE_ZROWS = (ACC_ROWS // NS) & ~7
ZREM = ACC_ROWS - NS * TILE_ZROWS
_ZSIZES = [128] * (TILE_ZROWS // 128) + (
    [TILE_ZROWS % 128] if TILE_ZROWS % 128 else [])


def _spmm_body(tab_hbm, meta_hbm, ewp_hbm, out_hbm,
               mb0, mb1, mb2, mb3, eb0, eb1, eb2, eb3, r0, r1, r2, acc,
               ms0, ms1, ms2, ms3, gs0, gs1, gs2, ss0, ss1, ss2):
    c = lax.axis_index("c")
    s = lax.axis_index("s")
    row0 = s * SP_ROWS_T
    coff = c * NP

    mslots = ((mb0, ms0, eb0), (mb1, ms1, eb1), (mb2, ms2, eb2), (mb3, ms3, eb3))
    rbufs = ((r0, gs0, ss0), (r1, gs1, ss1), (r2, gs2, ss2))

    def start_meta(ch, m):
        mb, ms, eb = mslots[m]
        pltpu.async_copy(meta_hbm.at[row0 + ch], mb, ms)
        pltpu.async_copy(ewp_hbm.at[row0 + ch], eb, ms)

    def wait_meta_fix(m):
        mb, ms, eb = mslots[m]
        pltpu.make_async_copy(meta_hbm.at[row0], mb, ms).wait()
        pltpu.make_async_copy(ewp_hbm.at[row0], eb, ms).wait()

        @plsc.parallel_loop(0, 8)
        def _fix(j):
            sl = pl.ds(j * 16, 16)
            mb[0, sl] = mb[0, sl] + coff

    def start_gather(m, b):
        mb = mslots[m][0]
        rb, gs, _ = rbufs[b]
        pltpu.async_copy(tab_hbm.at[mb.at[0]], rb, gs)

    def wait_gather(b):
        rb, gs, _ = rbufs[b]
        pltpu.make_async_copy(tab_hbm.at[mb0.at[0]], rb, gs).wait()

    def start_scatter(m, b):
        mb = mslots[m][0]
        rb, _, ss = rbufs[b]
        pltpu.async_copy(rb, acc.at[mb.at[1]], ss, add=True)

    def wait_scatter(b):
        rb, _, ss = rbufs[b]
        pltpu.make_async_copy(rb, acc.at[mb0.at[1]], ss).wait()

    def scale(m, b):
        eb = mslots[m][2]
        rb = rbufs[b][0]

        @plsc.parallel_loop(0, 128, 2)
        def _sc(e):
            w = lax.shift_right_logical(e, 1)
            bits = plsc.load_gather(eb, [jnp.full((16,), w, jnp.int32)])
            bclo = plsc.bitcast(lax.shift_left(bits, 16), jnp.float32)
            bchi = plsc.bitcast(
                jnp.bitwise_and(bits, jnp.int32(-65536)), jnp.float32)
            for j in range(HH // 16):
                sl = pl.ds(j * 16, 16)
                rb[e, sl] = rb[e, sl] * bclo
                rb[e + 1, sl] = rb[e + 1, sl] * bchi

    def zrow(r, carry):
        for j in range(HH // 16):
            r0[r, pl.ds(j * 16, 16)] = _zeros16()
        return carry

    lax.fori_loop(0, 128, zrow, 0)
    zbase = s * TILE_ZROWS
    zoff = 0
    for zn in _ZSIZES:
        pltpu.sync_copy(r0.at[pl.ds(0, zn)], acc.at[pl.ds(zbase + zoff, zn)])
        zoff += zn
    if ZREM:
        @pl.when(s == 0)
        def _zrem():
            pltpu.sync_copy(r0.at[pl.ds(0, ZREM)],
                            acc.at[pl.ds(NS * TILE_ZROWS, ZREM)])
    plsc.subcore_barrier()

    def step(ch, m, b):
        wait_gather(b)
        scale(m, b)
        start_scatter(m, b)
        if not (isinstance(ch, int) and ch == 0):
            wait_scatter((b + 2) % 3)
        if not (isinstance(ch, int) and ch + 2 >= SP_ROWS_T):
            wait_meta_fix((m + 2) % 4)
            start_gather((m + 2) % 4, (b + 2) % 3)
        if not (isinstance(ch, int) and ch + 3 >= SP_ROWS_T):
            start_meta(ch + 3, (m + 3) % 4)

    for m in range(3):
        start_meta(m, m)
    wait_meta_fix(0)
    start_gather(0, 0)
    wait_meta_fix(1)
    start_gather(1, 1)
    for ch in range(4):
        step(ch, ch % 4, ch % 3)

    def twelve(i, carry):
        ch0 = 4 + i * 12
        for k in range(12):
            step(ch0 + k, (4 + k) % 4, (4 + k) % 3)
        return carry

    lax.fori_loop(0, (SP_ROWS_T - 16) // 12, twelve, 0)

    for ch in range(SP_ROWS_T - 12, SP_ROWS_T):
        step(ch, ch % 4, ch % 3)
    wait_scatter((SP_ROWS_T - 1) % 3)

    plsc.subcore_barrier()
    zoff = 0
    for zn in _ZSIZES:
        sl = pl.ds(zbase + zoff, zn)
        pltpu.sync_copy(acc.at[sl], out_hbm.at[c, sl])
        zoff += zn
    if ZREM:
        @pl.when(s == 0)
        def _orem():
            sl = pl.ds(NS * TILE_ZROWS, ZREM)
            pltpu.sync_copy(acc.at[sl], out_hbm.at[c, sl])


_spmm_call = pl.kernel(
    _spmm_body,
    out_type=jax.ShapeDtypeStruct((NC, ACC_ROWS, HH), jnp.float32),
    mesh=plsc.VectorSubcoreMesh(core_axis_name="c", subcore_axis_name="s",
                                num_cores=NC, num_subcores=NS),
    scratch_types=[
        pltpu.VMEM((2, 128), jnp.int32),
        pltpu.VMEM((2, 128), jnp.int32),
        pltpu.VMEM((2, 128), jnp.int32),
        pltpu.VMEM((2, 128), jnp.int32),
        pltpu.VMEM((64,), jnp.int32),
        pltpu.VMEM((64,), jnp.int32),
        pltpu.VMEM((64,), jnp.int32),
        pltpu.VMEM((64,), jnp.int32),
        pltpu.VMEM((128, HH), jnp.float32),
        pltpu.VMEM((128, HH), jnp.float32),
        pltpu.VMEM((128, HH), jnp.float32),
        pltpu.VMEM_SHARED((ACC_ROWS, HH), jnp.float32),
        pltpu.SemaphoreType.DMA,
        pltpu.SemaphoreType.DMA,
        pltpu.SemaphoreType.DMA,
        pltpu.SemaphoreType.DMA,
        pltpu.SemaphoreType.DMA,
        pltpu.SemaphoreType.DMA,
        pltpu.SemaphoreType.DMA,
        pltpu.SemaphoreType.DMA,
        pltpu.SemaphoreType.DMA,
        pltpu.SemaphoreType.DMA,
    ],
    compiler_params=pltpu.CompilerParams(needs_layout_passes=False),
)


def _dinv_col(d0_ref, d1_ref):
    dsum = d0_ref[...] + d1_ref[...] + 1.0
    dinv = lax.rsqrt(dsum)
    return dinv.reshape(MB, 1)


def _tc1_body(x_ref, w1_ref, d0_ref, d1_ref, s_ref, self_ref):
    dc = _dinv_col(d0_ref, d1_ref)
    p = jnp.dot(x_ref[...], w1_ref[...], preferred_element_type=jnp.float32)
    s_ref[0] = p * dc
    self_ref[...] = p * (dc * dc)


_tc1_call = pl.pallas_call(
    _tc1_body,
    grid=(NMB, 2),
    in_specs=[
        pl.BlockSpec((MB, D), lambda m, h: (m, 0)),
        pl.BlockSpec((D, HH), lambda m, h: (0, h)),
        pl.BlockSpec((1, MB), lambda m, h: (0, m)),
        pl.BlockSpec((1, MB), lambda m, h: (0, m)),
    ],
    out_specs=[
        pl.BlockSpec((1, MB, HH), lambda m, h: (h, m, 0)),
        pl.BlockSpec((MB, HH), lambda m, h: (m, h)),
    ],
    out_shape=[
        jax.ShapeDtypeStruct((NC, NP, HH), jnp.float32),
        jax.ShapeDtypeStruct((NP, H), jnp.float32),
    ],
)


def _tc2_body(a_ref, self_ref, b1_ref, d0_ref, d1_ref, w2_ref, s_ref, self2_ref):
    dc = _dinv_col(d0_ref, d1_ref)
    afull = jnp.concatenate([a_ref[0], a_ref[1]], axis=1)
    z = afull * dc + self_ref[...] + b1_ref[...]
    h1 = jnp.maximum(z, 0.0)
    p = jnp.dot(h1, w2_ref[...], preferred_element_type=jnp.float32)
    s_ref[0] = p * dc
    self2_ref[...] = p * (dc * dc)


_tc2_call = pl.pallas_call(
    _tc2_body,
    grid=(NMB, 2),
    in_specs=[
        pl.BlockSpec((NC, MB, HH), lambda m, h: (0, m, 0)),
        pl.BlockSpec((MB, H), lambda m, h: (m, 0)),
        pl.BlockSpec((1, H), lambda m, h: (0, 0)),
        pl.BlockSpec((1, MB), lambda m, h: (0, m)),
        pl.BlockSpec((1, MB), lambda m, h: (0, m)),
        pl.BlockSpec((H, HH), lambda m, h: (0, h)),
    ],
    out_specs=[
        pl.BlockSpec((1, MB, HH), lambda m, h: (h, m, 0)),
        pl.BlockSpec((MB, HH), lambda m, h: (m, h)),
    ],
    out_shape=[
        jax.ShapeDtypeStruct((NC, NP, HH), jnp.float32),
        jax.ShapeDtypeStruct((NP, H), jnp.float32),
    ],
)


def _tc3_body(a_ref, self2_ref, b2_ref, d0_ref, d1_ref, bt_ref, wl_ref,
              bl_ref, o_ref, pool_ref):
    m = pl.program_id(0)
    dc = _dinv_col(d0_ref, d1_ref)
    afull = jnp.concatenate([a_ref[0], a_ref[1]], axis=1)
    z = afull * dc + self2_ref[...] + b2_ref[...]
    h2 = jnp.maximum(z, 0.0)

    @pl.when(m == 0)
    def _():
        pool_ref[...] = jnp.zeros((G, H), jnp.float32)

    bt = bt_ref[...]
    glo = bt[0, 0]
    ghi = bt[0, MB - 1]
    valid = (lax.broadcasted_iota(jnp.int32, (MB, 1), 0) + m * MB) < N
    validf = jnp.where(valid, 1.0, 0.0)

    def gbody(g, carry):
        maskf = jnp.where(bt == g, 1.0, 0.0).reshape(MB, 1) * validf
        red = jnp.max(h2 * maskf, axis=0, keepdims=True)
        cur = pool_ref[pl.ds(g, 1), :]
        pool_ref[pl.ds(g, 1), :] = jnp.maximum(cur, red)
        return carry

    lax.fori_loop(glo, ghi + 1, gbody, 0)

    @pl.when(m == NMB - 1)
    def _():
        o_ref[...] = (
            jnp.dot(pool_ref[...], wl_ref[...], preferred_element_type=jnp.float32)
            + bl_ref[...]
        )


_tc3_call = pl.pallas_call(
    _tc3_body,
    grid=(NMB,),
    in_specs=[
        pl.BlockSpec((NC, MB, HH), lambda m: (0, m, 0)),
        pl.BlockSpec((MB, H), lambda m: (m, 0)),
        pl.BlockSpec((1, H), lambda m: (0, 0)),
        pl.BlockSpec((1, MB), lambda m: (0, m)),
        pl.BlockSpec((1, MB), lambda m: (0, m)),
        pl.BlockSpec((1, MB), lambda m: (0, m)),
        pl.BlockSpec((H, C), lambda m: (0, 0)),
        pl.BlockSpec((1, C), lambda m: (0, 0)),
    ],
    out_specs=pl.BlockSpec((G, C), lambda m: (0, 0)),
    out_shape=jax.ShapeDtypeStruct((G, C), jnp.float32),
    scratch_shapes=[pltpu.VMEM((G, H), jnp.float32)],
    compiler_params=pltpu.CompilerParams(
        dimension_semantics=("arbitrary",),
    ),
)


def kernel(x, edge_index, edge_weight, batch, W1, b1, W2, b2, Wl, bl):
    pe = E_PAD - E
    srcp = jnp.pad(edge_index[0], (0, pe)).reshape(-1, 128)
    dstp = jnp.pad(edge_index[1], (0, pe)).reshape(-1, 128)
    ewp = jnp.pad(edge_weight, (0, pe)).reshape(-1, 128)

    emeta = jnp.stack([srcp, dstp], axis=1)
    ewpk = lax.bitcast_convert_type(
        ewp.astype(jnp.bfloat16).reshape(-1, 64, 2), jnp.int32)

    degp = _deg_call(dstp, ewp)
    d0 = degp[0:1, :]
    d1 = degp[1:2, :]

    xp = jnp.pad(x, ((0, NP - N), (0, 0)))
    batp = jnp.pad(batch, (0, NP - N), constant_values=G - 1).reshape(1, NP)

    s1cat, self1 = _tc1_call(xp, W1, d0, d1)
    acc1 = _spmm_call(s1cat.reshape(NC * NP, HH), emeta, ewpk)
    acc1 = jnp.pad(acc1, ((0, 0), (0, NP - ACC_ROWS), (0, 0)))

    s2cat, self2 = _tc2_call(acc1, self1, b1.reshape(1, H), d0, d1, W2)
    acc2 = _spmm_call(s2cat.reshape(NC * NP, HH), emeta, ewpk)
    acc2 = jnp.pad(acc2, ((0, 0), (0, NP - ACC_ROWS), (0, 0)))

    out = _tc3_call(acc2, self2, b2.reshape(1, H), d0, d1,
                    batp, Wl, bl.reshape(1, C))
    return out
